# baseline (speedup 1.0000x reference)
import functools

import jax
import jax.numpy as jnp
from jax.experimental import pallas as pl
from jax.experimental.pallas import tpu as pltpu

_EPS = 1e-5


def _prep_kernel(x_ref, w1_ref, b1_ref, w2_ref, b2_ref, ws_ref, bs_ref,
                 gs_ref, betas_ref, wf_ref, bf_ref,
                 sw_ref, wcat_ref, bcat_ref, gcat_ref, betacat_ref,
                 wfc_ref, bfc_ref, *, n_rows, n_strategies, d):
    gr = jnp.sum(x_ref[...], axis=0, keepdims=True) * (1.0 / n_rows)
    h = jnp.maximum(
        jnp.dot(gr, w1_ref[...], preferred_element_type=jnp.float32)
        + b1_ref[...], 0.0)
    logits = (jnp.dot(h, w2_ref[...], preferred_element_type=jnp.float32)
              + b2_ref[...])
    m = jnp.max(logits, axis=-1, keepdims=True)
    e = jnp.exp(logits - m)
    sm = e / jnp.sum(e, axis=-1, keepdims=True)
    sw_ref[...] = jnp.zeros_like(sw_ref)
    sw_ref[0:1, 0:n_strategies] = sm

    for s in range(n_strategies):
        sl = slice(s * d, (s + 1) * d)
        w = ws_ref[s]
        wcat_ref[:, sl] = w - jnp.mean(w, axis=1, keepdims=True)
        b = bs_ref[s:s + 1, :]
        bcat_ref[0:1, sl] = b - jnp.mean(b)
        sw_s = sm[0:1, s:s + 1]
        gcat_ref[0:1, sl] = gs_ref[s:s + 1, :] * sw_s
        betacat_ref[0:1, sl] = betas_ref[s:s + 1, :] * sw_s
    wf = wf_ref[...]
    wfc_ref[...] = wf - jnp.mean(wf, axis=1, keepdims=True)
    bfc_ref[...] = bf_ref[...] - jnp.mean(bf_ref[...])


def _main_kernel(x_ref, wcat_ref, bcat_ref, gcat_ref, betacat_ref,
                 wf_ref, bf_ref, gf_ref, bf2_ref, out_ref,
                 *, n_strategies, d):
    xb = x_ref[...]
    h = (jnp.dot(xb, wcat_ref[...], preferred_element_type=jnp.float32)
         + bcat_ref[...])
    acc = jnp.zeros_like(xb)
    inv_d = 1.0 / d
    for s in range(n_strategies):
        sl = slice(s * d, (s + 1) * d)
        hs = h[:, sl]
        ss = jnp.sum(hs * hs, axis=-1, keepdims=True) * inv_d
        r = jax.lax.rsqrt(ss + _EPS)
        acc = acc + jnp.maximum(hs * r * gcat_ref[:, sl] + betacat_ref[:, sl],
                                0.0)
    y = (jnp.dot(acc, wf_ref[...], preferred_element_type=jnp.float32)
         + bf_ref[...])
    ss = jnp.sum(y * y, axis=-1, keepdims=True) * inv_d
    r = jax.lax.rsqrt(ss + _EPS)
    out_ref[...] = jnp.maximum(y * r * gf_ref[...] + bf2_ref[...], 0.0)


def kernel(x, edge_index, W1, b1, W2, b2, Ws, bs, gs, betas, Wf, bf, gf, bf2):
    del edge_index
    n, d = x.shape
    s = Ws.shape[0]
    dh = W1.shape[1]
    lanes = 128

    f32 = jnp.float32
    sw, wcat, bcat, gcat, betacat, wfc, bfc = pl.pallas_call(
        functools.partial(_prep_kernel, n_rows=float(n), n_strategies=s, d=d),
        out_shape=(
            jax.ShapeDtypeStruct((1, lanes), f32),
            jax.ShapeDtypeStruct((d, s * d), f32),
            jax.ShapeDtypeStruct((1, s * d), f32),
            jax.ShapeDtypeStruct((1, s * d), f32),
            jax.ShapeDtypeStruct((1, s * d), f32),
            jax.ShapeDtypeStruct((d, d), f32),
            jax.ShapeDtypeStruct((1, d), f32),
        ),
    )(x, W1, b1.reshape(1, dh), W2, b2.reshape(1, s), Ws, bs, gs, betas,
      Wf, bf.reshape(1, d))
    del sw

    blk = 5000
    assert n % blk == 0
    nb = n // blk
    const1 = lambda i: (0, 0)

    out = pl.pallas_call(
        functools.partial(_main_kernel, n_strategies=s, d=d),
        grid=(nb,),
        in_specs=[
            pl.BlockSpec((blk, d), lambda i: (i, 0)),
            pl.BlockSpec((d, s * d), const1),
            pl.BlockSpec((1, s * d), const1),
            pl.BlockSpec((1, s * d), const1),
            pl.BlockSpec((1, s * d), const1),
            pl.BlockSpec((d, d), const1),
            pl.BlockSpec((1, d), const1),
            pl.BlockSpec((1, d), const1),
            pl.BlockSpec((1, d), const1),
        ],
        out_specs=pl.BlockSpec((blk, d), lambda i: (i, 0)),
        out_shape=jax.ShapeDtypeStruct((n, d), x.dtype),
        compiler_params=pltpu.CompilerParams(
            dimension_semantics=("arbitrary",),
        ),
    )(x, wcat, bcat, gcat, betacat, wfc, bfc,
      gf.reshape(1, d), bf2.reshape(1, d))
    return out


# speedup vs baseline: 1.0570x; 1.0570x over previous
import functools

import jax
import jax.numpy as jnp
from jax.experimental import pallas as pl
from jax.experimental.pallas import tpu as pltpu

_EPS = 1e-5


def _prep_kernel(x_ref, w1_ref, b1_ref, w2_ref, b2_ref, ws_ref, bs_ref,
                 gs_ref, betas_ref, wf_ref, bf_ref,
                 sw_ref, wcat_ref, bcat_ref, gcat_ref, betacat_ref,
                 wfc_ref, bfc_ref, *, n_rows, n_strategies, d):
    n_int = x_ref.shape[0]
    n_chunks = 8
    chunk = n_int // n_chunks
    parts = [jnp.sum(x_ref[k * chunk:(k + 1) * chunk, :], axis=0,
                     keepdims=True)
             for k in range(n_chunks)]
    if n_int % n_chunks:
        parts.append(jnp.sum(x_ref[n_chunks * chunk:, :], axis=0,
                             keepdims=True))
    while len(parts) > 1:
        parts = [parts[j] + parts[j + 1] for j in range(0, len(parts) - 1, 2)
                 ] + ([parts[-1]] if len(parts) % 2 else [])
    total = parts[0]
    gr = total * (1.0 / n_rows)
    h = jnp.maximum(
        jnp.dot(gr, w1_ref[...], preferred_element_type=jnp.float32)
        + b1_ref[...], 0.0)
    logits = (jnp.dot(h, w2_ref[...], preferred_element_type=jnp.float32)
              + b2_ref[...])
    m = jnp.max(logits, axis=-1, keepdims=True)
    e = jnp.exp(logits - m)
    sm = e / jnp.sum(e, axis=-1, keepdims=True)
    sw_ref[...] = jnp.zeros_like(sw_ref)
    sw_ref[0:1, 0:n_strategies] = sm

    for s in range(n_strategies):
        sl = slice(s * d, (s + 1) * d)
        w = ws_ref[s]
        wcat_ref[:, sl] = w - jnp.mean(w, axis=1, keepdims=True)
        b = bs_ref[s:s + 1, :]
        bcat_ref[0:1, sl] = b - jnp.mean(b)
        sw_s = sm[0:1, s:s + 1]
        gcat_ref[0:1, sl] = gs_ref[s:s + 1, :] * sw_s
        betacat_ref[0:1, sl] = betas_ref[s:s + 1, :] * sw_s
    wf = wf_ref[...]
    wfc_ref[...] = wf - jnp.mean(wf, axis=1, keepdims=True)
    bfc_ref[...] = bf_ref[...] - jnp.mean(bf_ref[...])


def _main_kernel(x_ref, wcat_ref, bcat_ref, gcat_ref, betacat_ref,
                 wf_ref, bf_ref, gf_ref, bf2_ref, out_ref,
                 *, n_strategies, d):
    xb = x_ref[...]
    h = (jnp.dot(xb, wcat_ref[...], preferred_element_type=jnp.float32)
         + bcat_ref[...])
    acc = jnp.zeros_like(xb)
    inv_d = 1.0 / d
    for s in range(n_strategies):
        sl = slice(s * d, (s + 1) * d)
        hs = h[:, sl]
        ss = jnp.sum(hs * hs, axis=-1, keepdims=True) * inv_d
        r = jax.lax.rsqrt(ss + _EPS)
        acc = acc + jnp.maximum(hs * r * gcat_ref[:, sl] + betacat_ref[:, sl],
                                0.0)
    y = (jnp.dot(acc, wf_ref[...], preferred_element_type=jnp.float32)
         + bf_ref[...])
    ss = jnp.sum(y * y, axis=-1, keepdims=True) * inv_d
    r = jax.lax.rsqrt(ss + _EPS)
    out_ref[...] = jnp.maximum(y * r * gf_ref[...] + bf2_ref[...], 0.0)


def kernel(x, edge_index, W1, b1, W2, b2, Ws, bs, gs, betas, Wf, bf, gf, bf2):
    del edge_index
    n, d = x.shape
    s = Ws.shape[0]
    dh = W1.shape[1]
    lanes = 128

    f32 = jnp.float32
    sw, wcat, bcat, gcat, betacat, wfc, bfc = pl.pallas_call(
        functools.partial(_prep_kernel, n_rows=float(n), n_strategies=s, d=d),
        out_shape=(
            jax.ShapeDtypeStruct((1, lanes), f32),
            jax.ShapeDtypeStruct((d, s * d), f32),
            jax.ShapeDtypeStruct((1, s * d), f32),
            jax.ShapeDtypeStruct((1, s * d), f32),
            jax.ShapeDtypeStruct((1, s * d), f32),
            jax.ShapeDtypeStruct((d, d), f32),
            jax.ShapeDtypeStruct((1, d), f32),
        ),
    )(x, W1, b1.reshape(1, dh), W2, b2.reshape(1, s), Ws, bs, gs, betas,
      Wf, bf.reshape(1, d))
    del sw

    blk = 2000
    assert n % blk == 0
    nb = n // blk
    const1 = lambda i: (0, 0)

    out = pl.pallas_call(
        functools.partial(_main_kernel, n_strategies=s, d=d),
        grid=(nb,),
        in_specs=[
            pl.BlockSpec((blk, d), lambda i: (i, 0)),
            pl.BlockSpec((d, s * d), const1),
            pl.BlockSpec((1, s * d), const1),
            pl.BlockSpec((1, s * d), const1),
            pl.BlockSpec((1, s * d), const1),
            pl.BlockSpec((d, d), const1),
            pl.BlockSpec((1, d), const1),
            pl.BlockSpec((1, d), const1),
            pl.BlockSpec((1, d), const1),
        ],
        out_specs=pl.BlockSpec((blk, d), lambda i: (i, 0)),
        out_shape=jax.ShapeDtypeStruct((n, d), x.dtype),
        compiler_params=pltpu.CompilerParams(
            dimension_semantics=("arbitrary",),
        ),
    )(x, wcat, bcat, gcat, betacat, wfc, bfc,
      gf.reshape(1, d), bf2.reshape(1, d))
    return out
